# baseline (speedup 1.0000x reference)
import functools

import jax
import jax.numpy as jnp
from jax import lax
from jax.experimental import pallas as pl
from jax.experimental.pallas import tpu as pltpu
from jax.experimental.pallas import tpu_sc as plsc

_NBUF = 4
_RBLK = 512


def _pad_table(Wt):
    D, V = Wt.shape
    nblk = pl.cdiv(V, _RBLK)

    def body(wt_ref, out_ref):
        x = wt_ref[...]
        y = jnp.transpose(x, (1, 0))
        z = jnp.concatenate([y, jnp.zeros((_RBLK, 128 - D), jnp.float32)], axis=1)
        out_ref[...] = z

    return pl.pallas_call(
        body,
        grid=(nblk,),
        in_specs=[pl.BlockSpec((D, _RBLK), lambda i: (0, i))],
        out_specs=pl.BlockSpec((_RBLK, 128), lambda i: (i, 0)),
        out_shape=jax.ShapeDtypeStruct((V, 128), jnp.float32),
    )(Wt)


def _make_gather(S, T, V, D, NC, NS):
    NW = NC * NS
    s_per_w = S // NW
    nbuf = _NBUF
    n_outer = s_per_w // nbuf
    mesh = plsc.VectorSubcoreMesh(core_axis_name="c", subcore_axis_name="s")

    @functools.partial(
        pl.kernel,
        mesh=mesh,
        out_type=jax.ShapeDtypeStruct((S, T, 2 * D), jnp.float32),
        scratch_types=[
            pltpu.VMEM((s_per_w * T,), jnp.int32),
            pltpu.VMEM((nbuf * T, 2 * D), jnp.float32),
            [pltpu.SemaphoreType.DMA] * _NBUF,
            [pltpu.SemaphoreType.DMA] * _NBUF,
        ],
    )
    def gather_kernel(table_hbm, idx_hbm, out_hbm, idx_v, rows_v, gsems, ssems):
        wid = lax.axis_index("s") * NC + lax.axis_index("c")
        base = wid * s_per_w
        pltpu.sync_copy(idx_hbm.at[pl.ds(base * T, s_per_w * T)], idx_v)

        def start_gather(j, b):
            pltpu.async_copy(
                table_hbm.at[idx_v.at[pl.ds(j * T, T)]],
                rows_v.at[pl.ds(b * T, T)],
                gsems[b],
            )

        def wait_gather(b):
            pltpu.make_async_copy(
                table_hbm.at[idx_v.at[pl.ds(0, T)]],
                rows_v.at[pl.ds(0, T)],
                gsems[b],
            ).wait()

        def start_store(j, b):
            pltpu.async_copy(
                rows_v.at[pl.ds(b * T, T)],
                out_hbm.at[base + j],
                ssems[b],
            )

        def wait_store(b):
            pltpu.make_async_copy(
                rows_v.at[pl.ds(0, T)], out_hbm.at[base], ssems[b]
            ).wait()

        for b in range(nbuf):
            start_gather(b, b)

        def outer(go, carry):
            for b in range(nbuf):
                j = go * nbuf + b
                wait_gather(b)
                start_store(j, b)
                wait_store(b)
                start_gather(j + nbuf, b)
            return carry

        lax.fori_loop(0, n_outer - 1, outer, 0)

        for b in range(nbuf):
            j = (n_outer - 1) * nbuf + b
            wait_gather(b)
            start_store(j, b)
        for b in range(nbuf):
            wait_store(b)

    return gather_kernel


def kernel(token_ids, W):
    S, T = token_ids.shape
    V, D = W.shape
    info = plsc.get_sparse_core_info()
    NC, NS = info.num_cores, info.num_subcores
    Wp = _pad_table(W.T)
    idx_flat = token_ids.reshape(S * T)
    out_wide = _make_gather(S, T, V, D, NC, NS)(Wp, idx_flat)
    return out_wide[:, :, :D]


# speedup vs baseline: 2.0485x; 2.0485x over previous
import functools

import jax
import jax.numpy as jnp
from jax import lax
from jax.experimental import pallas as pl
from jax.experimental.pallas import tpu as pltpu
from jax.experimental.pallas import tpu_sc as plsc

_NBUF = 4
_RBLK = 4096


def _pad_table(Wt):
    D, V = Wt.shape
    nblk = pl.cdiv(V, _RBLK)

    def body(wt_ref, out_ref):
        x = wt_ref[...]
        y = jnp.transpose(x, (1, 0))
        z = jnp.concatenate([y, jnp.zeros((_RBLK, 128 - D), jnp.float32)], axis=1)
        out_ref[...] = z

    return pl.pallas_call(
        body,
        grid=(nblk,),
        compiler_params=pltpu.CompilerParams(
            dimension_semantics=("arbitrary",)
        ),
        in_specs=[pl.BlockSpec((D, _RBLK), lambda i: (0, i))],
        out_specs=pl.BlockSpec((_RBLK, 128), lambda i: (i, 0)),
        out_shape=jax.ShapeDtypeStruct((V, 128), jnp.float32),
    )(Wt)


def _make_gather(S, T, V, D, NC, NS):
    NW = NC * NS
    s_per_w = S // NW
    nbuf = _NBUF
    n_outer = s_per_w // nbuf
    mesh = plsc.VectorSubcoreMesh(core_axis_name="c", subcore_axis_name="s")

    @functools.partial(
        pl.kernel,
        mesh=mesh,
        out_type=jax.ShapeDtypeStruct((S, T, 2 * D), jnp.float32),
        scratch_types=[
            pltpu.VMEM((s_per_w * T,), jnp.int32),
            pltpu.VMEM((nbuf * T, 2 * D), jnp.float32),
            [pltpu.SemaphoreType.DMA] * _NBUF,
            [pltpu.SemaphoreType.DMA] * _NBUF,
        ],
    )
    def gather_kernel(table_hbm, idx_hbm, out_hbm, idx_v, rows_v, gsems, ssems):
        wid = lax.axis_index("s") * NC + lax.axis_index("c")
        base = wid * s_per_w
        pltpu.sync_copy(idx_hbm.at[pl.ds(base * T, s_per_w * T)], idx_v)

        def start_gather(j, b):
            pltpu.async_copy(
                table_hbm.at[idx_v.at[pl.ds(j * T, T)]],
                rows_v.at[pl.ds(b * T, T)],
                gsems[b],
            )

        def wait_gather(b):
            pltpu.make_async_copy(
                table_hbm.at[idx_v.at[pl.ds(0, T)]],
                rows_v.at[pl.ds(0, T)],
                gsems[b],
            ).wait()

        def start_store(j, b):
            pltpu.async_copy(
                rows_v.at[pl.ds(b * T, T)],
                out_hbm.at[base + j],
                ssems[b],
            )

        def wait_store(b):
            pltpu.make_async_copy(
                rows_v.at[pl.ds(0, T)], out_hbm.at[base], ssems[b]
            ).wait()

        for b in range(nbuf):
            start_gather(b, b)

        def outer(go, carry):
            for b in range(nbuf):
                j = go * nbuf + b
                wait_gather(b)
                start_store(j, b)
                wait_store(b)
                start_gather(j + nbuf, b)
            return carry

        lax.fori_loop(0, n_outer - 1, outer, 0)

        for b in range(nbuf):
            j = (n_outer - 1) * nbuf + b
            wait_gather(b)
            start_store(j, b)
        for b in range(nbuf):
            wait_store(b)

    return gather_kernel


def kernel(token_ids, W):
    S, T = token_ids.shape
    V, D = W.shape
    info = plsc.get_sparse_core_info()
    NC, NS = info.num_cores, info.num_subcores
    Wp = _pad_table(W.T)
    idx_flat = token_ids.reshape(S * T)
    out_wide = _make_gather(S, T, V, D, NC, NS)(Wp, idx_flat)
    return out_wide[:, :, :D]


# speedup vs baseline: 2.2412x; 1.0941x over previous
import functools

import jax
import jax.numpy as jnp
from jax import lax
from jax.experimental import pallas as pl
from jax.experimental.pallas import tpu as pltpu
from jax.experimental.pallas import tpu_sc as plsc

_NBUF = 4
_RBLK = 8192


def _pad_table(Wt):
    D, V = Wt.shape
    nblk = pl.cdiv(V, _RBLK)

    def body(wt_ref, out_ref):
        x = wt_ref[...]
        y = jnp.transpose(x, (1, 0))
        z = jnp.concatenate([y, jnp.zeros((_RBLK, 128 - D), jnp.float32)], axis=1)
        out_ref[...] = z

    return pl.pallas_call(
        body,
        grid=(nblk,),
        compiler_params=pltpu.CompilerParams(
            dimension_semantics=("arbitrary",)
        ),
        in_specs=[pl.BlockSpec((D, _RBLK), lambda i: (0, i))],
        out_specs=pl.BlockSpec((_RBLK, 128), lambda i: (i, 0)),
        out_shape=jax.ShapeDtypeStruct((V, 128), jnp.float32),
    )(Wt)


def _make_gather(S, T, V, D, NC, NS):
    NW = NC * NS
    s_per_w = S // NW
    nbuf = _NBUF
    n_outer = s_per_w // nbuf
    mesh = plsc.VectorSubcoreMesh(core_axis_name="c", subcore_axis_name="s")

    @functools.partial(
        pl.kernel,
        mesh=mesh,
        out_type=jax.ShapeDtypeStruct((S, T, 2 * D), jnp.float32),
        scratch_types=[
            pltpu.VMEM((s_per_w * T,), jnp.int32),
            pltpu.VMEM((nbuf * T, 2 * D), jnp.float32),
            [pltpu.SemaphoreType.DMA] * _NBUF,
            [pltpu.SemaphoreType.DMA] * _NBUF,
        ],
    )
    def gather_kernel(table_hbm, idx_hbm, out_hbm, idx_v, rows_v, gsems, ssems):
        wid = lax.axis_index("s") * NC + lax.axis_index("c")
        base = wid * s_per_w
        pltpu.sync_copy(idx_hbm.at[pl.ds(base * T, s_per_w * T)], idx_v)

        def start_gather(j, b):
            pltpu.async_copy(
                table_hbm.at[idx_v.at[pl.ds(j * T, T)]],
                rows_v.at[pl.ds(b * T, T)],
                gsems[b],
            )

        def wait_gather(b):
            pltpu.make_async_copy(
                table_hbm.at[idx_v.at[pl.ds(0, T)]],
                rows_v.at[pl.ds(0, T)],
                gsems[b],
            ).wait()

        def start_store(j, b):
            pltpu.async_copy(
                rows_v.at[pl.ds(b * T, T)],
                out_hbm.at[base + j],
                ssems[b],
            )

        def wait_store(b):
            pltpu.make_async_copy(
                rows_v.at[pl.ds(0, T)], out_hbm.at[base], ssems[b]
            ).wait()

        for b in range(nbuf):
            start_gather(b, b)

        def outer(go, carry):
            for b in range(nbuf):
                j = go * nbuf + b
                wait_gather(b)
                start_store(j, b)
                wait_store(b)
                start_gather(j + nbuf, b)
            return carry

        lax.fori_loop(0, n_outer - 1, outer, 0)

        for b in range(nbuf):
            j = (n_outer - 1) * nbuf + b
            wait_gather(b)
            start_store(j, b)
        for b in range(nbuf):
            wait_store(b)

    return gather_kernel


def kernel(token_ids, W):
    S, T = token_ids.shape
    V, D = W.shape
    info = plsc.get_sparse_core_info()
    NC, NS = info.num_cores, info.num_subcores
    Wp = _pad_table(W.T)
    idx_flat = token_ids.reshape(S * T)
    out_wide = _make_gather(S, T, V, D, NC, NS)(Wp, idx_flat)
    return out_wide[:, :, :D]


# speedup vs baseline: 2.3066x; 1.0292x over previous
import functools

import jax
import jax.numpy as jnp
from jax import lax
from jax.experimental import pallas as pl
from jax.experimental.pallas import tpu as pltpu
from jax.experimental.pallas import tpu_sc as plsc

_NBUF = 4
_CH = 200
_RBLK = 16384


def _pad_table(Wt):
    D, V = Wt.shape
    nblk = pl.cdiv(V, _RBLK)

    def body(wt_ref, out_ref):
        x = wt_ref[...]
        y = jnp.transpose(x, (1, 0))
        z = jnp.concatenate([y, jnp.zeros((_RBLK, 128 - D), jnp.float32)], axis=1)
        out_ref[...] = z

    return pl.pallas_call(
        body,
        grid=(nblk,),
        compiler_params=pltpu.CompilerParams(
            dimension_semantics=("arbitrary",)
        ),
        in_specs=[pl.BlockSpec((D, _RBLK), lambda i: (0, i))],
        out_specs=pl.BlockSpec((_RBLK, 128), lambda i: (i, 0)),
        out_shape=jax.ShapeDtypeStruct((V, 128), jnp.float32),
    )(Wt)


def _make_gather(S, T, V, D, NC, NS):
    NW = NC * NS
    s_per_w = S // NW
    nbuf = _NBUF
    ch = _CH
    n_chunks = s_per_w * T // ch
    n_outer = n_chunks // nbuf
    mesh = plsc.VectorSubcoreMesh(core_axis_name="c", subcore_axis_name="s")

    @functools.partial(
        pl.kernel,
        mesh=mesh,
        out_type=jax.ShapeDtypeStruct((S, T, 2 * D), jnp.float32),
        scratch_types=[
            pltpu.VMEM((s_per_w * T,), jnp.int32),
            pltpu.VMEM((nbuf * ch, 2 * D), jnp.float32),
            [pltpu.SemaphoreType.DMA] * _NBUF,
            [pltpu.SemaphoreType.DMA] * _NBUF,
        ],
    )
    def gather_kernel(table_hbm, idx_hbm, out_hbm, idx_v, rows_v, gsems, ssems):
        wid = lax.axis_index("s") * NC + lax.axis_index("c")
        base = wid * s_per_w
        pltpu.sync_copy(idx_hbm.at[pl.ds(base * T, s_per_w * T)], idx_v)

        def start_gather(j, b):
            pltpu.async_copy(
                table_hbm.at[idx_v.at[pl.ds(j * ch, ch)]],
                rows_v.at[pl.ds(b * ch, ch)],
                gsems[b],
            )

        def wait_gather(b):
            pltpu.make_async_copy(
                table_hbm.at[idx_v.at[pl.ds(0, ch)]],
                rows_v.at[pl.ds(0, ch)],
                gsems[b],
            ).wait()

        def start_store(j, b):
            s = base + j * ch // T
            h = j * ch % T
            pltpu.async_copy(
                rows_v.at[pl.ds(b * ch, ch)],
                out_hbm.at[s].at[pl.ds(h, ch)],
                ssems[b],
            )

        def wait_store(b):
            pltpu.make_async_copy(
                rows_v.at[pl.ds(0, ch)],
                out_hbm.at[base].at[pl.ds(0, ch)],
                ssems[b],
            ).wait()

        for b in range(nbuf):
            start_gather(b, b)

        def outer(go, carry):
            for b in range(nbuf):
                j = go * nbuf + b
                wait_gather(b)
                start_store(j, b)
                wait_store(b)
                start_gather(j + nbuf, b)
            return carry

        lax.fori_loop(0, n_outer - 1, outer, 0)

        for b in range(nbuf):
            j = (n_outer - 1) * nbuf + b
            wait_gather(b)
            start_store(j, b)
        for b in range(nbuf):
            wait_store(b)

    return gather_kernel


def kernel(token_ids, W):
    S, T = token_ids.shape
    V, D = W.shape
    info = plsc.get_sparse_core_info()
    NC, NS = info.num_cores, info.num_subcores
    Wp = _pad_table(W.T)
    idx_flat = token_ids.reshape(S * T)
    out_wide = _make_gather(S, T, V, D, NC, NS)(Wp, idx_flat)
    return out_wide[:, :, :D]
